# Initial kernel scaffold; baseline (speedup 1.0000x reference)
import jax
import jax.numpy as jnp
from jax.experimental import pallas as pl


def kernel(bbox_pred, cls_logits, priors):
    raise NotImplementedError("write your pallas kernel here")


# speedup vs baseline: 10.4667x; 10.4667x over previous
import functools

import jax
import jax.numpy as jnp
from jax import lax
from jax.experimental import pallas as pl
from jax.experimental.pallas import tpu as pltpu

CENTER_VARIANCE = 0.1
SIZE_VARIANCE = 0.2
IMAGE_SIZE = 300.0
IOU_THRESH = 0.7
KEEP = 100
NEG = -1e30


def _prep_kernel(n_valid, cls_ref, loc_ref, pri_ref, out_ref):
    cls = cls_ref[0]
    xm = jnp.max(cls, axis=0, keepdims=True)
    unn = jnp.exp(cls - xm)
    den = jnp.sum(unn, axis=0, keepdims=True)
    soft = unn / den
    conf = jnp.max(soft, axis=0, keepdims=True)
    ciota = lax.broadcasted_iota(jnp.int32, soft.shape, 0)
    lab = jnp.min(jnp.where(soft == conf, ciota, soft.shape[0]),
                  axis=0, keepdims=True).astype(jnp.float32)

    loc = loc_ref[0]
    pr = pri_ref[...]
    l0, l1, l2, l3 = loc[0:1], loc[1:2], loc[2:3], loc[3:4]
    p0, p1, p2, p3 = pr[0:1], pr[1:2], pr[2:3], pr[3:4]
    cx = l0 * CENTER_VARIANCE * p2 + p0
    cy = l1 * CENTER_VARIANCE * p3 + p1
    w = jnp.exp(l2 * SIZE_VARIANCE) * p2
    h = jnp.exp(l3 * SIZE_VARIANCE) * p3
    x1 = (cx - w / 2.0) * IMAGE_SIZE
    y1 = (cy - h / 2.0) * IMAGE_SIZE
    x2 = (cx + w / 2.0) * IMAGE_SIZE
    y2 = (cy + h / 2.0) * IMAGE_SIZE

    lane = lax.broadcasted_iota(jnp.int32, (1, x1.shape[1]), 1)
    valid = lane < n_valid

    def vmax(a):
        return jnp.max(jnp.where(valid, a, float("-inf")))

    mc = jnp.maximum(jnp.maximum(vmax(x1), vmax(y1)),
                     jnp.maximum(vmax(x2), vmax(y2))) + 1.0
    off = lab * mc
    ox1 = x1 + off
    oy1 = y1 + off
    ox2 = x2 + off
    oy2 = y2 + off
    areas = jnp.clip(ox2 - ox1, 0.0) * jnp.clip(oy2 - oy1, 0.0)

    z = 0.0
    out_ref[0, 0:1, :] = jnp.where(valid, conf, NEG)
    out_ref[0, 1:2, :] = jnp.where(valid, ox1, z)
    out_ref[0, 2:3, :] = jnp.where(valid, oy1, z)
    out_ref[0, 3:4, :] = jnp.where(valid, ox2, z)
    out_ref[0, 4:5, :] = jnp.where(valid, oy2, z)
    out_ref[0, 5:6, :] = jnp.where(valid, areas, z)
    out_ref[0, 6:7, :] = jnp.where(valid, x1, z)
    out_ref[0, 7:8, :] = jnp.where(valid, y1, z)
    out_ref[0, 8:9, :] = jnp.where(valid, x2, z)
    out_ref[0, 9:10, :] = jnp.where(valid, y2, z)


def _nms_kernel(rows, in_ref, idx_ref, box_ref, sc_ref):
    R = rows
    sc_ref[...] = in_ref[0, 0:R, :]
    iota2 = (lax.broadcasted_iota(jnp.int32, (R, 128), 0) * 128
             + lax.broadcasted_iota(jnp.int32, (R, 128), 1))
    lane = lax.broadcasted_iota(jnp.int32, (1, 128), 1)

    OX1 = in_ref[0, 1 * R:2 * R, :]
    OY1 = in_ref[0, 2 * R:3 * R, :]
    OX2 = in_ref[0, 3 * R:4 * R, :]
    OY2 = in_ref[0, 4 * R:5 * R, :]
    AR = in_ref[0, 5 * R:6 * R, :]

    def body(i, carry):
        sel, b0, b1, b2, b3 = carry
        sc = sc_ref[...]
        m = jnp.max(sc)
        idx = jnp.min(jnp.where(sc == m, iota2, jnp.int32(2 ** 30)))
        r = idx // 128
        c = idx - r * 128
        lmask = lane == c

        def pick(sec):
            row = in_ref[0, pl.ds(sec * R + r, 1), :]
            return jnp.sum(jnp.where(lmask, row, 0.0))

        sx1 = pick(1)
        sy1 = pick(2)
        sx2 = pick(3)
        sy2 = pick(4)
        a1 = pick(5)

        xx1 = jnp.maximum(sx1, OX1)
        yy1 = jnp.maximum(sy1, OY1)
        xx2 = jnp.minimum(sx2, OX2)
        yy2 = jnp.minimum(sy2, OY2)
        iw = jnp.clip(xx2 - xx1, 0.0)
        ih = jnp.clip(yy2 - yy1, 0.0)
        inter = iw * ih
        iou = inter / (a1 + AR - inter + 1e-9)

        nsc = jnp.where(iou > IOU_THRESH, NEG, sc)
        nsc = jnp.where(iota2 == idx, NEG, nsc)
        sc_ref[...] = nsc

        im = lane == i
        sel = jnp.where(im, idx, sel)
        b0 = jnp.where(im, pick(6), b0)
        b1 = jnp.where(im, pick(7), b1)
        b2 = jnp.where(im, pick(8), b2)
        b3 = jnp.where(im, pick(9), b3)
        return (sel, b0, b1, b2, b3)

    z32 = jnp.zeros((1, 128), jnp.int32)
    zf = jnp.zeros((1, 128), jnp.float32)
    sel, b0, b1, b2, b3 = lax.fori_loop(0, KEEP, body, (z32, zf, zf, zf, zf))
    idx_ref[0, 0:1, :] = sel
    box_ref[0, 0:1, :] = b0
    box_ref[0, 1:2, :] = b1
    box_ref[0, 2:3, :] = b2
    box_ref[0, 3:4, :] = b3


def kernel(bbox_pred, cls_logits, priors):
    B, N, C = cls_logits.shape
    R = ((N + 127) // 128 + 7) // 8 * 8
    NPAD = R * 128

    clsp = jnp.pad(cls_logits, ((0, 0), (0, NPAD - N), (0, 0))).transpose(0, 2, 1)
    locp = jnp.pad(bbox_pred, ((0, 0), (0, NPAD - N), (0, 0))).transpose(0, 2, 1)
    prip = jnp.pad(priors, ((0, NPAD - N), (0, 0))).T

    prep = pl.pallas_call(
        functools.partial(_prep_kernel, N),
        grid=(B,),
        in_specs=[
            pl.BlockSpec((1, C, NPAD), lambda b: (b, 0, 0)),
            pl.BlockSpec((1, 4, NPAD), lambda b: (b, 0, 0)),
            pl.BlockSpec((4, NPAD), lambda b: (0, 0)),
        ],
        out_specs=pl.BlockSpec((1, 10, NPAD), lambda b: (b, 0, 0)),
        out_shape=jax.ShapeDtypeStruct((B, 10, NPAD), jnp.float32),
        compiler_params=pltpu.CompilerParams(
            dimension_semantics=("parallel",)),
    )(clsp, locp, prip)

    packed = prep.reshape(B, 10 * R, 128)

    idxo, boxo = pl.pallas_call(
        functools.partial(_nms_kernel, R),
        grid=(B,),
        in_specs=[pl.BlockSpec((1, 10 * R, 128), lambda b: (b, 0, 0))],
        out_specs=[
            pl.BlockSpec((1, 1, 128), lambda b: (b, 0, 0)),
            pl.BlockSpec((1, 4, 128), lambda b: (b, 0, 0)),
        ],
        out_shape=[
            jax.ShapeDtypeStruct((B, 1, 128), jnp.int32),
            jax.ShapeDtypeStruct((B, 4, 128), jnp.float32),
        ],
        scratch_shapes=[pltpu.VMEM((R, 128), jnp.float32)],
        compiler_params=pltpu.CompilerParams(
            dimension_semantics=("parallel",)),
    )(packed)

    nms_indices = idxo[:, 0, :KEEP]
    nms_boxes = boxo[:, :, :KEEP].transpose(0, 2, 1)
    return nms_boxes, nms_indices
